# baseline (speedup 1.0000x reference)
import functools

import jax
import jax.numpy as jnp
from jax import lax
from jax.experimental import pallas as pl
from jax.experimental.pallas import tpu as pltpu
from jax.experimental.pallas import tpu_sc as plsc


_SC_CHUNK = 128


def _gather_rows(table, gidx, base_row, n_rows):
    R, D = table.shape
    info = plsc.get_sparse_core_info()
    nw = info.num_cores * info.num_subcores
    per_w = n_rows // nw
    C = _SC_CHUNK
    iters = per_w // C
    assert per_w % C == 0 and n_rows % nw == 0

    assert iters % 2 == 0
    mesh = plsc.VectorSubcoreMesh(core_axis_name="c", subcore_axis_name="s")

    @functools.partial(
        pl.kernel,
        mesh=mesh,
        out_type=jax.ShapeDtypeStruct((n_rows, D), jnp.float32),
        scratch_types=[
            pltpu.VMEM((2, C), jnp.int32),
            pltpu.VMEM((2, C, D), jnp.float32),
            pltpu.SemaphoreType.DMA,
            pltpu.SemaphoreType.DMA,
        ],
    )
    def k(table_hbm, idx_hbm, out_hbm, idx_v, rows_v, sem0, sem1):
        wid = lax.axis_index("s") * info.num_cores + lax.axis_index("c")
        lbase = wid * per_w
        sems = (sem0, sem1)

        def fire(c, slot):
            off = lbase + c * C
            pltpu.sync_copy(idx_hbm.at[pl.ds(base_row + off, C)],
                            idx_v.at[slot])
            pltpu.async_copy(table_hbm.at[idx_v.at[slot]], rows_v.at[slot],
                             sems[slot])

        def drain(c, slot):
            pltpu.make_async_copy(table_hbm.at[idx_v.at[slot]],
                                  rows_v.at[slot], sems[slot]).wait()
            pltpu.sync_copy(rows_v.at[slot], out_hbm.at[pl.ds(lbase + c * C, C)])

        fire(0, 0)

        def body(i2, _):
            c0 = i2 * 2
            fire(c0 + 1, 1)
            drain(c0, 0)

            @pl.when(c0 + 2 < iters)
            def _():
                fire(c0 + 2, 0)

            drain(c0 + 1, 1)
            return 0

        lax.fori_loop(0, iters // 2, body, 0)

    return k(table, gidx)


_TL = 128


def _tc_body(*refs):
    (ef_ref, nf_ref, node_ref, mask_ref,
     w1e_ref, w1c_ref, w1n_ref, be1_ref, we2_ref, be2_ref,
     wn1a_ref, wn1b_ref, bn1_ref, wn2_ref, bn2_ref,
     lneg_ref, lneb_ref, lnng_ref, lnnb_ref) = refs[:19]
    eo_ref, no_ref = refs[-2], refs[-1]
    TL = _TL
    K, E = ef_ref.shape[1], ef_ref.shape[2]
    D = nf_ref.shape[3]
    TLK = TL * K
    f32 = jnp.float32

    ef = jnp.transpose(ef_ref[0], (0, 2, 1)).reshape(TLK, E)
    nf = nf_ref[0].reshape(TLK, D)
    node = node_ref[0]
    msk = mask_ref[0]

    h = jnp.dot(ef, w1e_ref[...], preferred_element_type=f32)
    h = h + jnp.dot(nf, w1n_ref[...], preferred_element_type=f32)
    c = jnp.dot(node, w1c_ref[...], preferred_element_type=f32)
    h = h.reshape(K, TL, 2 * E) + c[None, :, :] + be1_ref[...].reshape(1, 1, 2 * E)
    h = jnp.maximum(h, 0.0).reshape(TLK, 2 * E)
    eu = jnp.dot(h, we2_ref[...], preferred_element_type=f32) + be2_ref[...]
    er = ef + eu
    avg = jnp.full((E, E), 1.0 / E, dtype=f32)
    m = jnp.dot(er, avg, preferred_element_type=f32)
    d = er - m
    v = jnp.dot(d * d, avg, preferred_element_type=f32)
    en = d * lax.rsqrt(v + 1e-5) * lneg_ref[...] + lneb_ref[...]

    nfs = jnp.sum(nf.reshape(K, TL, D), axis=0)
    es = jnp.sum(en.reshape(K, TL, E), axis=0)

    g = (jnp.dot(nfs, wn1a_ref[...], preferred_element_type=f32)
         + jnp.dot(es, wn1b_ref[...], preferred_element_type=f32)
         + bn1_ref[...])
    g = jnp.maximum(g, 0.0)
    nu = jnp.dot(g, wn2_ref[...], preferred_element_type=f32) + bn2_ref[...]
    nr = node + nu
    m2 = jnp.mean(nr, axis=-1, keepdims=True)
    v2 = jnp.mean((nr - m2) ** 2, axis=-1, keepdims=True)
    nn = (nr - m2) * lax.rsqrt(v2 + 1e-5) * lnng_ref[...] + lnnb_ref[...]

    no_ref[0] = nn * msk
    enm = en.reshape(K, TL, E) * msk[None, :, :]
    eo_ref[0] = jnp.transpose(enm, (0, 2, 1))


def _tc_call(chunk, nbc, eft, nfc, node, mask3,
             w1e, w1c, w1n, be1, we2, be2,
             wn1a, wn1b, bn1, wn2, bn2, lneg, lneb, lnng, lnnb,
             eo_prev=None, no_prev=None):
    B, K, E, L = eft.shape
    D = node.shape[2]
    TL = _TL
    TLK = TL * K
    off = chunk * nbc

    def row3(bs, o):
        return pl.BlockSpec(bs, lambda b, i, o=o: (b + o, i, 0))

    def lcol4(bs, o):
        return pl.BlockSpec(bs, lambda b, i, o=o: (b + o, 0, 0, i))

    def nf4(bs):
        return pl.BlockSpec(bs, lambda b, i: (b, 0, i, 0))

    def full(a):
        return pl.BlockSpec(a.shape, lambda b, i: (0, 0))

    grid = (nbc, L // TL)
    in_specs = [
        lcol4((1, K, E, TL), off),
        nf4((1, K, TL, D)),
        row3((1, TL, D), off),
        row3((1, TL, 1), off),
        full(w1e), full(w1c), full(w1n), full(be1), full(we2), full(be2),
        full(wn1a), full(wn1b), full(bn1), full(wn2), full(bn2),
        full(lneg), full(lneb), full(lnng), full(lnnb),
    ]
    args = [eft, nfc, node, mask3, w1e, w1c, w1n, be1, we2, be2,
            wn1a, wn1b, bn1, wn2, bn2, lneg, lneb, lnng, lnnb]
    aliases = {}
    if eo_prev is not None:
        hbm = pl.BlockSpec(memory_space=pltpu.MemorySpace.HBM)
        in_specs += [hbm, hbm]
        args += [eo_prev, no_prev]
        aliases = {19: 0, 20: 1}

    eo, no = pl.pallas_call(
        _tc_body,
        grid=grid,
        in_specs=in_specs,
        out_specs=[lcol4((1, K, E, TL), off), row3((1, TL, D), off)],
        out_shape=[
            jax.ShapeDtypeStruct((B, K, E, L), jnp.float32),
            jax.ShapeDtypeStruct((B, L, D), jnp.float32),
        ],
        input_output_aliases=aliases,
    )(*args)
    return eo, no


def kernel(node_features, edge_features, neighbor_indices, mask,
           W_e1, b_e1, W_e2, b_e2, W_n1, b_n1, W_n2, b_n2,
           ln_eg, ln_eb, ln_ng, ln_nb):
    B, L, D = node_features.shape
    K = neighbor_indices.shape[2]
    E = edge_features.shape[3]

    idx = jnp.transpose(neighbor_indices.astype(jnp.int32), (0, 2, 1))
    gidx = (jnp.arange(B, dtype=jnp.int32)[:, None, None] * L + idx).reshape(-1)
    table = node_features.reshape(B * L, D)
    mask3 = mask.reshape(B, L, 1)
    eft = jnp.transpose(edge_features, (0, 2, 3, 1))

    w1e = W_e1[:E]
    w1c = W_e1[E:E + D]
    w1n = W_e1[E + D:]
    wn1a = W_n1[:D]
    wn1b = W_n1[D:]

    nchunks = 8
    nbc = B // nchunks
    rows_c = nbc * L * K

    nf_chunks = [
        _gather_rows(table, gidx, c * rows_c, rows_c).reshape(nbc, K, L, D)
        for c in range(nchunks)
    ]

    eo = no = None
    for c in range(nchunks):
        eo, no = _tc_call(
            c, nbc, eft, nf_chunks[c], node_features, mask3,
            w1e, w1c, w1n, b_e1.reshape(1, -1), W_e2, b_e2.reshape(1, -1),
            wn1a, wn1b, b_n1.reshape(1, -1), W_n2, b_n2.reshape(1, -1),
            ln_eg.reshape(1, -1), ln_eb.reshape(1, -1),
            ln_ng.reshape(1, -1), ln_nb.reshape(1, -1),
            eo_prev=eo, no_prev=no)

    return no, jnp.transpose(eo, (0, 3, 1, 2))


# speedup vs baseline: 1.0252x; 1.0252x over previous
import functools

import jax
import jax.numpy as jnp
from jax import lax
from jax.experimental import pallas as pl
from jax.experimental.pallas import tpu as pltpu
from jax.experimental.pallas import tpu_sc as plsc


_SC_CHUNK = 128


def _gather_rows(table, gidx, base_row, n_rows):
    R, D = table.shape
    info = plsc.get_sparse_core_info()
    nw = info.num_cores * info.num_subcores
    per_w = n_rows // nw
    C = _SC_CHUNK
    iters = per_w // C
    assert per_w % C == 0 and n_rows % nw == 0

    assert iters % 2 == 0
    mesh = plsc.VectorSubcoreMesh(core_axis_name="c", subcore_axis_name="s")

    @functools.partial(
        pl.kernel,
        mesh=mesh,
        out_type=jax.ShapeDtypeStruct((n_rows, D), jnp.float32),
        scratch_types=[
            pltpu.VMEM((2, C), jnp.int32),
            pltpu.VMEM((2, C, D), jnp.float32),
            pltpu.SemaphoreType.DMA,
            pltpu.SemaphoreType.DMA,
        ],
    )
    def k(table_hbm, idx_hbm, out_hbm, idx_v, rows_v, sem0, sem1):
        wid = lax.axis_index("s") * info.num_cores + lax.axis_index("c")
        lbase = wid * per_w
        sems = (sem0, sem1)

        def fire(c, slot):
            off = lbase + c * C
            pltpu.sync_copy(idx_hbm.at[pl.ds(base_row + off, C)],
                            idx_v.at[slot])
            pltpu.async_copy(table_hbm.at[idx_v.at[slot]], rows_v.at[slot],
                             sems[slot])

        def drain(c, slot):
            pltpu.make_async_copy(table_hbm.at[idx_v.at[slot]],
                                  rows_v.at[slot], sems[slot]).wait()
            pltpu.sync_copy(rows_v.at[slot], out_hbm.at[pl.ds(lbase + c * C, C)])

        fire(0, 0)

        def body(i2, _):
            c0 = i2 * 2
            fire(c0 + 1, 1)
            drain(c0, 0)

            @pl.when(c0 + 2 < iters)
            def _():
                fire(c0 + 2, 0)

            drain(c0 + 1, 1)
            return 0

        lax.fori_loop(0, iters // 2, body, 0)

    return k(table, gidx)


_TL = 128


def _tc_body(*refs):
    (ef_ref, nf_ref, node_ref, mask_ref,
     w1e_ref, w1c_ref, w1n_ref, be1_ref, we2_ref, be2_ref,
     wn1a_ref, wn1b_ref, bn1_ref, wn2_ref, bn2_ref,
     lneg_ref, lneb_ref, lnng_ref, lnnb_ref) = refs[:19]
    eo_ref, no_ref = refs[-2], refs[-1]
    TL = _TL
    K, E = ef_ref.shape[1], ef_ref.shape[2]
    D = nf_ref.shape[3]
    TLK = TL * K
    f32 = jnp.float32

    ef = jnp.transpose(ef_ref[0], (0, 2, 1)).reshape(TLK, E)
    nf = nf_ref[0].reshape(TLK, D)
    node = node_ref[0]
    msk = mask_ref[0]

    h = jnp.dot(ef, w1e_ref[...], preferred_element_type=f32)
    h = h + jnp.dot(nf, w1n_ref[...], preferred_element_type=f32)
    c = jnp.dot(node, w1c_ref[...], preferred_element_type=f32)
    h = h.reshape(K, TL, 2 * E) + c[None, :, :] + be1_ref[...].reshape(1, 1, 2 * E)
    h = jnp.maximum(h, 0.0).reshape(TLK, 2 * E)
    eu = jnp.dot(h, we2_ref[...], preferred_element_type=f32) + be2_ref[...]
    er = ef + eu
    avg = jnp.full((E, E), 1.0 / E, dtype=f32)
    m = jnp.dot(er, avg, preferred_element_type=f32)
    d = er - m
    v = jnp.dot(d * d, avg, preferred_element_type=f32)
    en = d * lax.rsqrt(v + 1e-5) * lneg_ref[...] + lneb_ref[...]

    nfs = jnp.sum(nf.reshape(K, TL, D), axis=0)
    es = jnp.sum(en.reshape(K, TL, E), axis=0)

    g = (jnp.dot(nfs, wn1a_ref[...], preferred_element_type=f32)
         + jnp.dot(es, wn1b_ref[...], preferred_element_type=f32)
         + bn1_ref[...])
    g = jnp.maximum(g, 0.0)
    nu = jnp.dot(g, wn2_ref[...], preferred_element_type=f32) + bn2_ref[...]
    nr = node + nu
    m2 = jnp.mean(nr, axis=-1, keepdims=True)
    v2 = jnp.mean((nr - m2) ** 2, axis=-1, keepdims=True)
    nn = (nr - m2) * lax.rsqrt(v2 + 1e-5) * lnng_ref[...] + lnnb_ref[...]

    no_ref[0] = nn * msk
    enm = en.reshape(K, TL, E) * msk[None, :, :]
    eo_ref[0] = jnp.transpose(enm, (0, 2, 1))


def _tc_call(chunk, nbc, eft, nfc, node, mask3,
             w1e, w1c, w1n, be1, we2, be2,
             wn1a, wn1b, bn1, wn2, bn2, lneg, lneb, lnng, lnnb,
             eo_prev=None, no_prev=None):
    B, K, E, L = eft.shape
    D = node.shape[2]
    TL = _TL
    TLK = TL * K
    off = chunk * nbc

    def row3(bs, o):
        return pl.BlockSpec(bs, lambda b, i, o=o: (b + o, i, 0))

    def lcol4(bs, o):
        return pl.BlockSpec(bs, lambda b, i, o=o: (b + o, 0, 0, i))

    def nf4(bs):
        return pl.BlockSpec(bs, lambda b, i: (b, 0, i, 0))

    def full(a):
        return pl.BlockSpec(a.shape, lambda b, i: (0, 0))

    grid = (nbc, L // TL)
    in_specs = [
        lcol4((1, K, E, TL), off),
        nf4((1, K, TL, nfc.shape[3])),
        row3((1, TL, D), off),
        row3((1, TL, 1), off),
        full(w1e), full(w1c), full(w1n), full(be1), full(we2), full(be2),
        full(wn1a), full(wn1b), full(bn1), full(wn2), full(bn2),
        full(lneg), full(lneb), full(lnng), full(lnnb),
    ]
    args = [eft, nfc, node, mask3, w1e, w1c, w1n, be1, we2, be2,
            wn1a, wn1b, bn1, wn2, bn2, lneg, lneb, lnng, lnnb]
    aliases = {}
    if eo_prev is not None:
        hbm = pl.BlockSpec(memory_space=pltpu.MemorySpace.HBM)
        in_specs += [hbm, hbm]
        args += [eo_prev, no_prev]
        aliases = {19: 0, 20: 1}

    eo, no = pl.pallas_call(
        _tc_body,
        grid=grid,
        in_specs=in_specs,
        out_specs=[lcol4((1, K, E, TL), off), row3((1, TL, D), off)],
        out_shape=[
            jax.ShapeDtypeStruct((B, K, E, L), jnp.float32),
            jax.ShapeDtypeStruct((B, L, D), jnp.float32),
        ],
        input_output_aliases=aliases,
    )(*args)
    return eo, no


def kernel(node_features, edge_features, neighbor_indices, mask,
           W_e1, b_e1, W_e2, b_e2, W_n1, b_n1, W_n2, b_n2,
           ln_eg, ln_eb, ln_ng, ln_nb):
    B, L, D = node_features.shape
    K = neighbor_indices.shape[2]
    E = edge_features.shape[3]

    idx = jnp.transpose(neighbor_indices.astype(jnp.int32), (0, 2, 1))
    gidx = (jnp.arange(B, dtype=jnp.int32)[:, None, None] * L + idx).reshape(-1)
    table = node_features.reshape(B * L, D)
    mask3 = mask.reshape(B, L, 1)
    eft = jnp.transpose(edge_features, (0, 2, 3, 1))

    w1e = W_e1[:E]
    w1c = W_e1[E:E + D]
    w1n = W_e1[E + D:]
    wn1a = W_n1[:D]
    wn1b = W_n1[D:]

    nchunks = 4
    nbc = B // nchunks
    rows_c = nbc * L * K

    nf_chunks = [
        _gather_rows(table, gidx, c * rows_c, rows_c).reshape(nbc, K, L, D)
        for c in range(nchunks)
    ]

    eo = no = None
    for c in range(nchunks):
        eo, no = _tc_call(
            c, nbc, eft, nf_chunks[c], node_features, mask3,
            w1e, w1c, w1n, b_e1.reshape(1, -1), W_e2, b_e2.reshape(1, -1),
            wn1a, wn1b, b_n1.reshape(1, -1), W_n2, b_n2.reshape(1, -1),
            ln_eg.reshape(1, -1), ln_eb.reshape(1, -1),
            ln_ng.reshape(1, -1), ln_nb.reshape(1, -1),
            eo_prev=eo, no_prev=no)

    return no, jnp.transpose(eo, (0, 3, 1, 2))


# speedup vs baseline: 1.1110x; 1.0837x over previous
import functools

import jax
import jax.numpy as jnp
from jax import lax
from jax.experimental import pallas as pl
from jax.experimental.pallas import tpu as pltpu
from jax.experimental.pallas import tpu_sc as plsc


_SC_CHUNK = 128


def _gather_rows(table, gidx, base_row, n_rows):
    R, D = table.shape
    info = plsc.get_sparse_core_info()
    nw = info.num_cores * info.num_subcores
    per_w = n_rows // nw
    C = _SC_CHUNK
    iters = per_w // C
    assert per_w % C == 0 and n_rows % nw == 0

    assert iters % 2 == 0
    mesh = plsc.VectorSubcoreMesh(core_axis_name="c", subcore_axis_name="s")

    @functools.partial(
        pl.kernel,
        mesh=mesh,
        out_type=jax.ShapeDtypeStruct((n_rows, D), jnp.float32),
        scratch_types=[
            pltpu.VMEM((2, C), jnp.int32),
            pltpu.VMEM((2, C, D), jnp.float32),
            pltpu.SemaphoreType.DMA,
            pltpu.SemaphoreType.DMA,
        ],
    )
    def k(table_hbm, idx_hbm, out_hbm, idx_v, rows_v, sem0, sem1):
        wid = lax.axis_index("s") * info.num_cores + lax.axis_index("c")
        lbase = wid * per_w
        sems = (sem0, sem1)

        def fire(c, slot):
            off = lbase + c * C
            pltpu.sync_copy(idx_hbm.at[pl.ds(base_row + off, C)],
                            idx_v.at[slot])
            pltpu.async_copy(table_hbm.at[idx_v.at[slot]], rows_v.at[slot],
                             sems[slot])

        def drain(c, slot):
            pltpu.make_async_copy(table_hbm.at[idx_v.at[slot]],
                                  rows_v.at[slot], sems[slot]).wait()
            pltpu.sync_copy(rows_v.at[slot], out_hbm.at[pl.ds(lbase + c * C, C)])

        fire(0, 0)

        def body(i2, _):
            c0 = i2 * 2
            fire(c0 + 1, 1)
            drain(c0, 0)

            @pl.when(c0 + 2 < iters)
            def _():
                fire(c0 + 2, 0)

            drain(c0 + 1, 1)
            return 0

        lax.fori_loop(0, iters // 2, body, 0)

    return k(table, gidx)


_TL = 256


def _tc_body(*refs):
    (ef_ref, nf_ref, node_ref, mask_ref,
     w1e_ref, w1c_ref, w1n_ref, be1_ref, we2_ref, be2_ref,
     wn1a_ref, wn1b_ref, bn1_ref, wn2_ref, bn2_ref,
     lneg_ref, lneb_ref, lnng_ref, lnnb_ref) = refs[:19]
    eo_ref, no_ref = refs[-2], refs[-1]
    TL = _TL
    K, E = ef_ref.shape[1], ef_ref.shape[2]
    D = nf_ref.shape[3]
    TLK = TL * K
    f32 = jnp.float32

    ef = jnp.transpose(ef_ref[0], (0, 2, 1)).reshape(TLK, E)
    nf = nf_ref[0].reshape(TLK, D)
    node = node_ref[0]
    msk = mask_ref[0]

    h = jnp.dot(ef, w1e_ref[...], preferred_element_type=f32)
    h = h + jnp.dot(nf, w1n_ref[...], preferred_element_type=f32)
    c = jnp.dot(node, w1c_ref[...], preferred_element_type=f32)
    h = h.reshape(K, TL, 2 * E) + c[None, :, :] + be1_ref[...].reshape(1, 1, 2 * E)
    h = jnp.maximum(h, 0.0).reshape(TLK, 2 * E)
    eu = jnp.dot(h, we2_ref[...], preferred_element_type=f32) + be2_ref[...]
    er = ef + eu
    avg = jnp.full((E, E), 1.0 / E, dtype=f32)
    m = jnp.dot(er, avg, preferred_element_type=f32)
    d = er - m
    v = jnp.dot(d * d, avg, preferred_element_type=f32)
    en = d * lax.rsqrt(v + 1e-5) * lneg_ref[...] + lneb_ref[...]

    nfs = jnp.sum(nf.reshape(K, TL, D), axis=0)
    es = jnp.sum(en.reshape(K, TL, E), axis=0)

    g = (jnp.dot(nfs, wn1a_ref[...], preferred_element_type=f32)
         + jnp.dot(es, wn1b_ref[...], preferred_element_type=f32)
         + bn1_ref[...])
    g = jnp.maximum(g, 0.0)
    nu = jnp.dot(g, wn2_ref[...], preferred_element_type=f32) + bn2_ref[...]
    nr = node + nu
    m2 = jnp.mean(nr, axis=-1, keepdims=True)
    v2 = jnp.mean((nr - m2) ** 2, axis=-1, keepdims=True)
    nn = (nr - m2) * lax.rsqrt(v2 + 1e-5) * lnng_ref[...] + lnnb_ref[...]

    no_ref[0] = nn * msk
    enm = en.reshape(K, TL, E) * msk[None, :, :]
    eo_ref[0] = jnp.transpose(enm, (0, 2, 1))


def _tc_call(chunk, nbc, eft, nfc, node, mask3,
             w1e, w1c, w1n, be1, we2, be2,
             wn1a, wn1b, bn1, wn2, bn2, lneg, lneb, lnng, lnnb,
             eo_prev=None, no_prev=None):
    B, K, E, L = eft.shape
    D = node.shape[2]
    TL = _TL
    TLK = TL * K
    off = chunk * nbc

    def row3(bs, o):
        return pl.BlockSpec(bs, lambda b, i, o=o: (b + o, i, 0))

    def lcol4(bs, o):
        return pl.BlockSpec(bs, lambda b, i, o=o: (b + o, 0, 0, i))

    def nf4(bs):
        return pl.BlockSpec(bs, lambda b, i: (b, 0, i, 0))

    def full(a):
        return pl.BlockSpec(a.shape, lambda b, i: (0, 0))

    grid = (nbc, L // TL)
    in_specs = [
        lcol4((1, K, E, TL), off),
        nf4((1, K, TL, nfc.shape[3])),
        row3((1, TL, D), off),
        row3((1, TL, 1), off),
        full(w1e), full(w1c), full(w1n), full(be1), full(we2), full(be2),
        full(wn1a), full(wn1b), full(bn1), full(wn2), full(bn2),
        full(lneg), full(lneb), full(lnng), full(lnnb),
    ]
    args = [eft, nfc, node, mask3, w1e, w1c, w1n, be1, we2, be2,
            wn1a, wn1b, bn1, wn2, bn2, lneg, lneb, lnng, lnnb]
    aliases = {}
    if eo_prev is not None:
        hbm = pl.BlockSpec(memory_space=pltpu.MemorySpace.HBM)
        in_specs += [hbm, hbm]
        args += [eo_prev, no_prev]
        aliases = {19: 0, 20: 1}

    eo, no = pl.pallas_call(
        _tc_body,
        grid=grid,
        in_specs=in_specs,
        out_specs=[lcol4((1, K, E, TL), off), row3((1, TL, D), off)],
        out_shape=[
            jax.ShapeDtypeStruct((B, K, E, L), jnp.float32),
            jax.ShapeDtypeStruct((B, L, D), jnp.float32),
        ],
        input_output_aliases=aliases,
    )(*args)
    return eo, no


def kernel(node_features, edge_features, neighbor_indices, mask,
           W_e1, b_e1, W_e2, b_e2, W_n1, b_n1, W_n2, b_n2,
           ln_eg, ln_eb, ln_ng, ln_nb):
    B, L, D = node_features.shape
    K = neighbor_indices.shape[2]
    E = edge_features.shape[3]

    idx = jnp.transpose(neighbor_indices.astype(jnp.int32), (0, 2, 1))
    gidx = (jnp.arange(B, dtype=jnp.int32)[:, None, None] * L + idx).reshape(-1)
    table = node_features.reshape(B * L, D)
    mask3 = mask.reshape(B, L, 1)
    eft = jnp.transpose(edge_features, (0, 2, 3, 1))

    w1e = W_e1[:E]
    w1c = W_e1[E:E + D]
    w1n = W_e1[E + D:]
    wn1a = W_n1[:D]
    wn1b = W_n1[D:]

    nchunks = 4
    nbc = B // nchunks
    rows_c = nbc * L * K

    nf_chunks = [
        _gather_rows(table, gidx, c * rows_c, rows_c).reshape(nbc, K, L, D)
        for c in range(nchunks)
    ]

    eo = no = None
    for c in range(nchunks):
        eo, no = _tc_call(
            c, nbc, eft, nf_chunks[c], node_features, mask3,
            w1e, w1c, w1n, b_e1.reshape(1, -1), W_e2, b_e2.reshape(1, -1),
            wn1a, wn1b, b_n1.reshape(1, -1), W_n2, b_n2.reshape(1, -1),
            ln_eg.reshape(1, -1), ln_eb.reshape(1, -1),
            ln_ng.reshape(1, -1), ln_nb.reshape(1, -1),
            eo_prev=eo, no_prev=no)

    return no, jnp.transpose(eo, (0, 3, 1, 2))


# speedup vs baseline: 1.1269x; 1.0143x over previous
import functools

import jax
import jax.numpy as jnp
from jax import lax
from jax.experimental import pallas as pl
from jax.experimental.pallas import tpu as pltpu
from jax.experimental.pallas import tpu_sc as plsc


_SC_CHUNK = 128


def _gather_rows(table, gidx, base_row, n_rows):
    R, D = table.shape
    info = plsc.get_sparse_core_info()
    nw = info.num_cores * info.num_subcores
    per_w = n_rows // nw
    C = _SC_CHUNK
    iters = per_w // C
    assert per_w % C == 0 and n_rows % nw == 0

    assert iters % 2 == 0
    mesh = plsc.VectorSubcoreMesh(core_axis_name="c", subcore_axis_name="s")

    @functools.partial(
        pl.kernel,
        mesh=mesh,
        out_type=jax.ShapeDtypeStruct((n_rows, D), jnp.float32),
        scratch_types=[
            pltpu.VMEM((2, C), jnp.int32),
            pltpu.VMEM((2, C, D), jnp.float32),
            pltpu.SemaphoreType.DMA,
            pltpu.SemaphoreType.DMA,
        ],
    )
    def k(table_hbm, idx_hbm, out_hbm, idx_v, rows_v, sem0, sem1):
        wid = lax.axis_index("s") * info.num_cores + lax.axis_index("c")
        lbase = wid * per_w
        sems = (sem0, sem1)

        def fire(c, slot):
            off = lbase + c * C
            pltpu.sync_copy(idx_hbm.at[pl.ds(base_row + off, C)],
                            idx_v.at[slot])
            pltpu.async_copy(table_hbm.at[idx_v.at[slot]], rows_v.at[slot],
                             sems[slot])

        def drain(c, slot):
            pltpu.make_async_copy(table_hbm.at[idx_v.at[slot]],
                                  rows_v.at[slot], sems[slot]).wait()
            pltpu.sync_copy(rows_v.at[slot], out_hbm.at[pl.ds(lbase + c * C, C)])

        fire(0, 0)

        def body(i2, _):
            c0 = i2 * 2
            fire(c0 + 1, 1)
            drain(c0, 0)

            @pl.when(c0 + 2 < iters)
            def _():
                fire(c0 + 2, 0)

            drain(c0 + 1, 1)
            return 0

        lax.fori_loop(0, iters // 2, body, 0)

    return k(table, gidx)


_TL = 512


def _tc_body(*refs):
    (ef_ref, nf_ref, node_ref, mask_ref,
     w1e_ref, w1c_ref, w1n_ref, be1_ref, we2_ref, be2_ref,
     wn1a_ref, wn1b_ref, bn1_ref, wn2_ref, bn2_ref,
     lneg_ref, lneb_ref, lnng_ref, lnnb_ref) = refs[:19]
    eo_ref, no_ref = refs[-2], refs[-1]
    TL = _TL
    K, E = ef_ref.shape[1], ef_ref.shape[2]
    D = nf_ref.shape[3]
    TLK = TL * K
    f32 = jnp.float32

    ef = jnp.transpose(ef_ref[0], (0, 2, 1)).reshape(TLK, E)
    nf = nf_ref[0].reshape(TLK, D)
    node = node_ref[0]
    msk = mask_ref[0]

    h = jnp.dot(ef, w1e_ref[...], preferred_element_type=f32)
    h = h + jnp.dot(nf, w1n_ref[...], preferred_element_type=f32)
    c = jnp.dot(node, w1c_ref[...], preferred_element_type=f32)
    h = h.reshape(K, TL, 2 * E) + c[None, :, :] + be1_ref[...].reshape(1, 1, 2 * E)
    h = jnp.maximum(h, 0.0).reshape(TLK, 2 * E)
    eu = jnp.dot(h, we2_ref[...], preferred_element_type=f32) + be2_ref[...]
    er = ef + eu
    avg = jnp.full((E, E), 1.0 / E, dtype=f32)
    m = jnp.dot(er, avg, preferred_element_type=f32)
    d = er - m
    v = jnp.dot(d * d, avg, preferred_element_type=f32)
    en = d * lax.rsqrt(v + 1e-5) * lneg_ref[...] + lneb_ref[...]

    nfs = jnp.sum(nf.reshape(K, TL, D), axis=0)
    es = jnp.sum(en.reshape(K, TL, E), axis=0)

    g = (jnp.dot(nfs, wn1a_ref[...], preferred_element_type=f32)
         + jnp.dot(es, wn1b_ref[...], preferred_element_type=f32)
         + bn1_ref[...])
    g = jnp.maximum(g, 0.0)
    nu = jnp.dot(g, wn2_ref[...], preferred_element_type=f32) + bn2_ref[...]
    nr = node + nu
    m2 = jnp.mean(nr, axis=-1, keepdims=True)
    v2 = jnp.mean((nr - m2) ** 2, axis=-1, keepdims=True)
    nn = (nr - m2) * lax.rsqrt(v2 + 1e-5) * lnng_ref[...] + lnnb_ref[...]

    no_ref[0] = nn * msk
    enm = en.reshape(K, TL, E) * msk[None, :, :]
    eo_ref[0] = jnp.transpose(enm, (0, 2, 1))


def _tc_call(chunk, nbc, eft, nfc, node, mask3,
             w1e, w1c, w1n, be1, we2, be2,
             wn1a, wn1b, bn1, wn2, bn2, lneg, lneb, lnng, lnnb,
             eo_prev=None, no_prev=None):
    B, K, E, L = eft.shape
    D = node.shape[2]
    TL = _TL
    TLK = TL * K
    off = chunk * nbc

    def row3(bs, o):
        return pl.BlockSpec(bs, lambda b, i, o=o: (b + o, i, 0))

    def lcol4(bs, o):
        return pl.BlockSpec(bs, lambda b, i, o=o: (b + o, 0, 0, i))

    def nf4(bs):
        return pl.BlockSpec(bs, lambda b, i: (b, 0, i, 0))

    def full(a):
        return pl.BlockSpec(a.shape, lambda b, i: (0, 0))

    grid = (nbc, L // TL)
    in_specs = [
        lcol4((1, K, E, TL), off),
        nf4((1, K, TL, nfc.shape[3])),
        row3((1, TL, D), off),
        row3((1, TL, 1), off),
        full(w1e), full(w1c), full(w1n), full(be1), full(we2), full(be2),
        full(wn1a), full(wn1b), full(bn1), full(wn2), full(bn2),
        full(lneg), full(lneb), full(lnng), full(lnnb),
    ]
    args = [eft, nfc, node, mask3, w1e, w1c, w1n, be1, we2, be2,
            wn1a, wn1b, bn1, wn2, bn2, lneg, lneb, lnng, lnnb]
    aliases = {}
    if eo_prev is not None:
        hbm = pl.BlockSpec(memory_space=pltpu.MemorySpace.HBM)
        in_specs += [hbm, hbm]
        args += [eo_prev, no_prev]
        aliases = {19: 0, 20: 1}

    eo, no = pl.pallas_call(
        _tc_body,
        grid=grid,
        in_specs=in_specs,
        out_specs=[lcol4((1, K, E, TL), off), row3((1, TL, D), off)],
        out_shape=[
            jax.ShapeDtypeStruct((B, K, E, L), jnp.float32),
            jax.ShapeDtypeStruct((B, L, D), jnp.float32),
        ],
        input_output_aliases=aliases,
    )(*args)
    return eo, no


def kernel(node_features, edge_features, neighbor_indices, mask,
           W_e1, b_e1, W_e2, b_e2, W_n1, b_n1, W_n2, b_n2,
           ln_eg, ln_eb, ln_ng, ln_nb):
    B, L, D = node_features.shape
    K = neighbor_indices.shape[2]
    E = edge_features.shape[3]

    idx = jnp.transpose(neighbor_indices.astype(jnp.int32), (0, 2, 1))
    gidx = (jnp.arange(B, dtype=jnp.int32)[:, None, None] * L + idx).reshape(-1)
    table = node_features.reshape(B * L, D)
    mask3 = mask.reshape(B, L, 1)
    eft = jnp.transpose(edge_features, (0, 2, 3, 1))

    w1e = W_e1[:E]
    w1c = W_e1[E:E + D]
    w1n = W_e1[E + D:]
    wn1a = W_n1[:D]
    wn1b = W_n1[D:]

    nchunks = 4
    nbc = B // nchunks
    rows_c = nbc * L * K

    nf_chunks = [
        _gather_rows(table, gidx, c * rows_c, rows_c).reshape(nbc, K, L, D)
        for c in range(nchunks)
    ]

    eo = no = None
    for c in range(nchunks):
        eo, no = _tc_call(
            c, nbc, eft, nf_chunks[c], node_features, mask3,
            w1e, w1c, w1n, b_e1.reshape(1, -1), W_e2, b_e2.reshape(1, -1),
            wn1a, wn1b, b_n1.reshape(1, -1), W_n2, b_n2.reshape(1, -1),
            ln_eg.reshape(1, -1), ln_eb.reshape(1, -1),
            ln_ng.reshape(1, -1), ln_nb.reshape(1, -1),
            eo_prev=eo, no_prev=no)

    return no, jnp.transpose(eo, (0, 3, 1, 2))
